# baseline (speedup 1.0000x reference)
import jax
import jax.numpy as jnp
from jax import lax
from jax.experimental import pallas as pl
from jax.experimental.pallas import tpu as pltpu
from jax.experimental.pallas import tpu_sc as plsc

_NC = 2
_NS = 16
_NW = _NC * _NS
_CH = 128


def _body(idx_hbm, table_hbm, out_hbm, idx_v, buf0, buf1, sem0, sem1):
    nch = idx_v.shape[0]
    wid = lax.axis_index("s") * _NC + lax.axis_index("c")
    base = wid * (nch * _CH)

    pltpu.sync_copy(idx_hbm.at[wid], idx_v)

    bufs = (buf0, buf1)
    sems = (sem0, sem1)

    pltpu.make_async_copy(table_hbm.at[idx_v.at[0]], buf0, sem0).start()

    def step(j, b):
        nb = (b + 1) % 2

        @pl.when(j + 1 < nch)
        def _():
            pltpu.make_async_copy(
                table_hbm.at[idx_v.at[j + 1]], bufs[nb], sems[nb]
            ).start()

        pltpu.make_async_copy(table_hbm.at[idx_v.at[j]], bufs[b], sems[b]).wait()
        pltpu.sync_copy(bufs[b], out_hbm.at[pl.ds(base + j * _CH, _CH)])

    def outer(i, carry):
        step(i * 2, 0)
        step(i * 2 + 1, 1)
        return carry

    lax.fori_loop(0, nch // 2, outer, 0)


def kernel(input_ids, table):
    b, l = input_ids.shape
    dim = table.shape[1]
    total = b * l
    per_w = total // _NW
    nch = per_w // _CH
    idx = input_ids.reshape(_NW, nch, _CH)

    mesh = plsc.VectorSubcoreMesh(
        core_axis_name="c", subcore_axis_name="s", num_cores=_NC, num_subcores=_NS
    )
    out = pl.kernel(
        _body,
        out_type=jax.ShapeDtypeStruct((total, dim), jnp.float32),
        mesh=mesh,
        scratch_types=[
            pltpu.VMEM((nch, _CH), jnp.int32),
            pltpu.VMEM((_CH, dim), jnp.float32),
            pltpu.VMEM((_CH, dim), jnp.float32),
            pltpu.SemaphoreType.DMA,
            pltpu.SemaphoreType.DMA,
        ],
        compiler_params=pltpu.CompilerParams(use_tc_tiling_on_sc=False),
    )(idx, table)
    return out.reshape(b, l, dim)


# speedup vs baseline: 1.0193x; 1.0193x over previous
import jax
import jax.numpy as jnp
from jax import lax
from jax.experimental import pallas as pl
from jax.experimental.pallas import tpu as pltpu
from jax.experimental.pallas import tpu_sc as plsc

_NC = 2
_NS = 16
_NW = _NC * _NS
_CH = 128


_NBUF = 4
_LOOK = 2


def _body(idx_hbm, table_hbm, out_hbm, idx_v, *scratch):
    bufs = scratch[:_NBUF]
    gsems = scratch[_NBUF : 2 * _NBUF]
    wsems = scratch[2 * _NBUF :]
    nch = idx_v.shape[0]
    wid = lax.axis_index("s") * _NC + lax.axis_index("c")
    base = wid * (nch * _CH)

    pltpu.sync_copy(idx_hbm.at[wid], idx_v)

    def gather(j, b):
        return pltpu.make_async_copy(table_hbm.at[idx_v.at[j]], bufs[b], gsems[b])

    def writeback(j, b):
        return pltpu.make_async_copy(
            bufs[b], out_hbm.at[pl.ds(base + j * _CH, _CH)], wsems[b]
        )

    for c in range(_LOOK):
        gather(c, c).start()

    def step(j, b):
        jn = j + _LOOK
        bn = (b + _LOOK) % _NBUF

        @pl.when(jn < nch)
        def _():
            @pl.when(j >= _NBUF - _LOOK)
            def _():
                writeback(jn - _NBUF, bn).wait()

            gather(jn, bn).start()

        gather(j, b).wait()
        writeback(j, b).start()

    def outer(i, carry):
        for b in range(_NBUF):
            step(i * _NBUF + b, b)
        return carry

    lax.fori_loop(0, nch // _NBUF, outer, 0)

    for c in range(nch - _NBUF, nch):
        writeback(c, c % _NBUF).wait()


def kernel(input_ids, table):
    b, l = input_ids.shape
    dim = table.shape[1]
    total = b * l
    per_w = total // _NW
    nch = per_w // _CH
    idx = input_ids.reshape(_NW, nch, _CH)

    mesh = plsc.VectorSubcoreMesh(
        core_axis_name="c", subcore_axis_name="s", num_cores=_NC, num_subcores=_NS
    )
    out = pl.kernel(
        _body,
        out_type=jax.ShapeDtypeStruct((total, dim), jnp.float32),
        mesh=mesh,
        scratch_types=(
            [pltpu.VMEM((nch, _CH), jnp.int32)]
            + [pltpu.VMEM((_CH, dim), jnp.float32)] * _NBUF
            + [pltpu.SemaphoreType.DMA] * (2 * _NBUF)
        ),
        compiler_params=pltpu.CompilerParams(use_tc_tiling_on_sc=False),
    )(idx, table)
    return out.reshape(b, l, dim)
